# baseline (speedup 1.0000x reference)
import jax
import jax.numpy as jnp
from jax.experimental import pallas as pl
from jax.experimental.pallas import tpu as pltpu

W_MAP = 0.3
W_ORI = 0.7
BLOCK = 1024


def _vq_block_kernel(z_ref, p_ref, zhat_ref, ztilde_ref, idx_ref):
    z = z_ref[...]
    P = p_ref[...]
    m = jax.lax.dot_general(
        z, P, (((1,), (1,)), ((), ())),
        preferred_element_type=jnp.float32)
    znorm = jnp.sum(z * z, axis=1, keepdims=True)
    pnorm = jnp.sum(P * P, axis=1)[None, :]
    dists = znorm + pnorm - 2.0 * m
    idx = jnp.argmin(dists, axis=1)
    k = dists.shape[1]
    onehot = (jax.lax.broadcasted_iota(jnp.int32, (z.shape[0], k), 1)
              == idx[:, None]).astype(jnp.float32)
    zt = jax.lax.dot_general(
        onehot, P, (((1,), (0,)), ((), ())),
        preferred_element_type=jnp.float32)
    ztilde_ref[...] = zt
    zhat_ref[...] = W_ORI * z + W_MAP * zt
    idx_ref[...] = idx[:, None]


def kernel(z, P):
    n, d = z.shape
    k = P.shape[0]
    grid = (n // BLOCK,)
    zhat, ztilde, idx2d = pl.pallas_call(
        _vq_block_kernel,
        grid=grid,
        in_specs=[
            pl.BlockSpec((BLOCK, d), lambda i: (i, 0)),
            pl.BlockSpec((k, d), lambda i: (0, 0)),
        ],
        out_specs=[
            pl.BlockSpec((BLOCK, d), lambda i: (i, 0)),
            pl.BlockSpec((BLOCK, d), lambda i: (i, 0)),
            pl.BlockSpec((BLOCK, 1), lambda i: (i, 0)),
        ],
        out_shape=[
            jax.ShapeDtypeStruct((n, d), jnp.float32),
            jax.ShapeDtypeStruct((n, d), jnp.float32),
            jax.ShapeDtypeStruct((n, 1), jnp.int32),
        ],
        compiler_params=pltpu.CompilerParams(
            dimension_semantics=("arbitrary",),
        ),
    )(z, P)
    return (zhat, ztilde, idx2d[:, 0])


# speedup vs baseline: 1.1469x; 1.1469x over previous
import jax
import jax.numpy as jnp
from jax.experimental import pallas as pl
from jax.experimental.pallas import tpu as pltpu

W_MAP = 0.3
W_ORI = 0.7
BLOCK = 2048


def _vq_block_kernel(z_ref, p_ref, zhat_ref, ztilde_ref, idx_ref):
    z = z_ref[...]
    P = p_ref[...]
    m = jax.lax.dot_general(
        z, P, (((1,), (1,)), ((), ())),
        preferred_element_type=jnp.float32)
    znorm = jnp.sum(z * z, axis=1, keepdims=True)
    pnorm = jnp.sum(P * P, axis=1)[None, :]
    dists = znorm + pnorm - 2.0 * m
    idx = jnp.argmin(dists, axis=1)
    k = dists.shape[1]
    onehot = (jax.lax.broadcasted_iota(jnp.int32, (z.shape[0], k), 1)
              == idx[:, None]).astype(jnp.float32)
    zt = jax.lax.dot_general(
        onehot, P, (((1,), (0,)), ((), ())),
        preferred_element_type=jnp.float32)
    ztilde_ref[...] = zt
    zhat_ref[...] = W_ORI * z + W_MAP * zt
    idx_ref[...] = idx[:, None]


def kernel(z, P):
    n, d = z.shape
    k = P.shape[0]
    grid = (n // BLOCK,)
    zhat, ztilde, idx2d = pl.pallas_call(
        _vq_block_kernel,
        grid=grid,
        in_specs=[
            pl.BlockSpec((BLOCK, d), lambda i: (i, 0)),
            pl.BlockSpec((k, d), lambda i: (0, 0)),
        ],
        out_specs=[
            pl.BlockSpec((BLOCK, d), lambda i: (i, 0)),
            pl.BlockSpec((BLOCK, d), lambda i: (i, 0)),
            pl.BlockSpec((BLOCK, 1), lambda i: (i, 0)),
        ],
        out_shape=[
            jax.ShapeDtypeStruct((n, d), jnp.float32),
            jax.ShapeDtypeStruct((n, d), jnp.float32),
            jax.ShapeDtypeStruct((n, 1), jnp.int32),
        ],
        compiler_params=pltpu.CompilerParams(
            dimension_semantics=("arbitrary",),
        ),
    )(z, P)
    return (zhat, ztilde, idx2d[:, 0])


# speedup vs baseline: 1.1579x; 1.0095x over previous
import jax
import jax.numpy as jnp
from jax.experimental import pallas as pl
from jax.experimental.pallas import tpu as pltpu

W_MAP = 0.3
W_ORI = 0.7
BLOCK = 2048


def _vq_block_kernel(z_ref, p_ref, zhat_ref, ztilde_ref, idx_ref):
    z = z_ref[...]
    P = p_ref[...]
    m = jax.lax.dot_general(
        z, P, (((1,), (1,)), ((), ())),
        preferred_element_type=jnp.float32)
    znorm = jnp.sum(z * z, axis=1, keepdims=True)
    pnorm = jnp.sum(P * P, axis=1)[None, :]
    dists = (znorm + pnorm) - 2.0 * m
    b, k = dists.shape
    minval = jnp.min(dists, axis=1, keepdims=True)
    iota = jax.lax.broadcasted_iota(jnp.int32, (b, k), 1)
    idx = jnp.min(jnp.where(dists == minval, iota, k), axis=1)
    onehot = (iota == idx[:, None]).astype(jnp.float32)
    zt = jax.lax.dot_general(
        onehot, P, (((1,), (0,)), ((), ())),
        preferred_element_type=jnp.float32)
    ztilde_ref[...] = zt
    zhat_ref[...] = W_ORI * z + W_MAP * zt
    idx_ref[...] = idx[:, None]


def kernel(z, P):
    n, d = z.shape
    k = P.shape[0]
    grid = (n // BLOCK,)
    zhat, ztilde, idx2d = pl.pallas_call(
        _vq_block_kernel,
        grid=grid,
        in_specs=[
            pl.BlockSpec((BLOCK, d), lambda i: (i, 0)),
            pl.BlockSpec((k, d), lambda i: (0, 0)),
        ],
        out_specs=[
            pl.BlockSpec((BLOCK, d), lambda i: (i, 0)),
            pl.BlockSpec((BLOCK, d), lambda i: (i, 0)),
            pl.BlockSpec((BLOCK, 1), lambda i: (i, 0)),
        ],
        out_shape=[
            jax.ShapeDtypeStruct((n, d), jnp.float32),
            jax.ShapeDtypeStruct((n, d), jnp.float32),
            jax.ShapeDtypeStruct((n, 1), jnp.int32),
        ],
        compiler_params=pltpu.CompilerParams(
            dimension_semantics=("arbitrary",),
        ),
    )(z, P)
    return (zhat, ztilde, idx2d[:, 0])
